# Initial kernel scaffold; baseline (speedup 1.0000x reference)
import jax
import jax.numpy as jnp
from jax.experimental import pallas as pl


def kernel(indices, table):
    raise NotImplementedError("write your pallas kernel here")


# speedup vs baseline: 1.8683x; 1.8683x over previous
import functools

import jax
import jax.numpy as jnp
from jax import lax
from jax.experimental import pallas as pl
from jax.experimental.pallas import tpu as pltpu
from jax.experimental.pallas import tpu_sc as plsc

BATCH = 16384
FIELDS = 26
EMBED_DIM = 32
N = BATCH * FIELDS
NUM_WORKERS = 32
PER_WORKER = N // NUM_WORKERS
CHUNK = 1024
NUM_CHUNKS = PER_WORKER // CHUNK


def kernel(indices, table):
    num_item = table.shape[0] - 1
    idx_flat = indices.reshape(-1).astype(jnp.int32)

    mesh = plsc.VectorSubcoreMesh(core_axis_name="c", subcore_axis_name="s")

    @functools.partial(
        pl.kernel,
        mesh=mesh,
        out_type=jax.ShapeDtypeStruct((N, EMBED_DIM), jnp.float32),
        scratch_types=[
            pltpu.VMEM((CHUNK,), jnp.int32),
            pltpu.VMEM((CHUNK, EMBED_DIM), jnp.float32),
            pltpu.SemaphoreType.DMA,
        ],
        compiler_params=pltpu.CompilerParams(use_tc_tiling_on_sc=False),
    )
    def gather_kernel(idx_hbm, table_hbm, out_hbm, idx_v, rows_v, sem):
        wid = lax.axis_index("s") * 2 + lax.axis_index("c")
        base = wid * PER_WORKER

        def chunk_body(c, carry):
            off = base + c * CHUNK
            pltpu.sync_copy(idx_hbm.at[pl.ds(off, CHUNK)], idx_v)

            def remap_body(i, carry2):
                v = idx_v[pl.ds(i * 16, 16)]
                idx_v[pl.ds(i * 16, 16)] = jnp.where(v == -1, num_item, v)
                return carry2

            lax.fori_loop(0, CHUNK // 16, remap_body, 0, unroll=4)
            pltpu.async_copy(table_hbm.at[idx_v], rows_v, sem).wait()
            pltpu.sync_copy(rows_v, out_hbm.at[pl.ds(off, CHUNK)])
            return carry

        lax.fori_loop(0, NUM_CHUNKS, chunk_body, 0)

    out = gather_kernel(idx_flat, table)
    return out.reshape(BATCH, FIELDS * EMBED_DIM)


# speedup vs baseline: 1.9078x; 1.0212x over previous
import functools

import jax
import jax.numpy as jnp
from jax import lax
from jax.experimental import pallas as pl
from jax.experimental.pallas import tpu as pltpu
from jax.experimental.pallas import tpu_sc as plsc

BATCH = 16384
FIELDS = 26
EMBED_DIM = 32
N = BATCH * FIELDS
NUM_WORKERS = 32
PER_WORKER = N // NUM_WORKERS
CHUNK = 1664
NUM_CHUNKS = PER_WORKER // CHUNK
NBUF = 2


def kernel(indices, table):
    num_item = table.shape[0] - 1
    idx_flat = indices.reshape(-1).astype(jnp.int32)

    mesh = plsc.VectorSubcoreMesh(core_axis_name="c", subcore_axis_name="s")

    @functools.partial(
        pl.kernel,
        mesh=mesh,
        out_type=jax.ShapeDtypeStruct((N, EMBED_DIM), jnp.float32),
        scratch_types=[
            pltpu.VMEM((NBUF, CHUNK), jnp.int32),
            pltpu.VMEM((NBUF, CHUNK, EMBED_DIM), jnp.float32),
            pltpu.SemaphoreType.DMA((NBUF,)),
            pltpu.SemaphoreType.DMA((NBUF,)),
        ],
        compiler_params=pltpu.CompilerParams(use_tc_tiling_on_sc=False),
    )
    def gather_kernel(idx_hbm, table_hbm, out_hbm, idx_v, rows_v, gsem, ssem):
        wid = lax.axis_index("s") * 2 + lax.axis_index("c")
        base = wid * PER_WORKER

        def start_gather(c):
            b = c % NBUF
            off = base + c * CHUNK
            pltpu.sync_copy(idx_hbm.at[pl.ds(off, CHUNK)], idx_v.at[b])

            def remap_body(i, carry):
                v = idx_v[b, pl.ds(i * 16, 16)]
                idx_v[b, pl.ds(i * 16, 16)] = jnp.where(v == -1, num_item, v)
                return carry

            lax.fori_loop(0, CHUNK // 16, remap_body, 0, unroll=8)
            return pltpu.async_copy(
                table_hbm.at[idx_v.at[b]], rows_v.at[b], gsem.at[b])

        def start_store(c):
            b = c % NBUF
            off = base + c * CHUNK
            return pltpu.async_copy(
                rows_v.at[b], out_hbm.at[pl.ds(off, CHUNK)], ssem.at[b])

        gh = [None] * NUM_CHUNKS
        sh = [None] * NUM_CHUNKS
        for c in range(NUM_CHUNKS):
            if c >= NBUF:
                sh[c - NBUF].wait()
            gh[c] = start_gather(c)
            if c >= 1:
                gh[c - 1].wait()
                sh[c - 1] = start_store(c - 1)
        gh[NUM_CHUNKS - 1].wait()
        sh[NUM_CHUNKS - 1] = start_store(NUM_CHUNKS - 1)
        sh[NUM_CHUNKS - 2].wait()
        sh[NUM_CHUNKS - 1].wait()

    out = gather_kernel(idx_flat, table)
    return out.reshape(BATCH, FIELDS * EMBED_DIM)


# speedup vs baseline: 1.9095x; 1.0009x over previous
import functools

import jax
import jax.numpy as jnp
from jax import lax
from jax.experimental import pallas as pl
from jax.experimental.pallas import tpu as pltpu
from jax.experimental.pallas import tpu_sc as plsc
from jax.experimental.layout import Layout, with_layout_constraint

BATCH = 16384
FIELDS = 26
EMBED_DIM = 32
N = BATCH * FIELDS
NUM_WORKERS = 32
PER_WORKER = N // NUM_WORKERS
CHUNK = 1664
NUM_CHUNKS = PER_WORKER // CHUNK
NBUF = 2

VOCAB1 = 1000001
TBLK = 2048
NBLK = (VOCAB1 + TBLK - 1) // TBLK
VPAD = NBLK * TBLK


def _repack_block(src_ref, dst_ref):
    x = src_ref[...]
    z = x.T.reshape(TBLK // 4, 4, EMBED_DIM)
    dst_ref[...] = jnp.concatenate([z[:, g, :] for g in range(4)], axis=1)


def _repack_table(table):
    tt = table.T
    return pl.pallas_call(
        _repack_block,
        grid=(NBLK,),
        in_specs=[pl.BlockSpec((EMBED_DIM, TBLK), lambda k: (0, k))],
        out_specs=pl.BlockSpec((TBLK // 4, 128), lambda k: (k, 0)),
        out_shape=jax.ShapeDtypeStruct((VPAD // 4, 128), jnp.float32),
    )(tt)


def kernel(indices, table):
    num_item = table.shape[0] - 1
    idx_flat = indices.reshape(-1).astype(jnp.int32)

    t128 = _repack_table(table)
    table_lin = with_layout_constraint(
        t128.reshape(VPAD, EMBED_DIM),
        Layout(major_to_minor=(0, 1), tiling=((8,), (1024,))))

    mesh = plsc.VectorSubcoreMesh(core_axis_name="c", subcore_axis_name="s")

    @functools.partial(
        pl.kernel,
        mesh=mesh,
        out_type=jax.ShapeDtypeStruct((N, EMBED_DIM), jnp.float32),
        scratch_types=[
            pltpu.VMEM((NBUF, CHUNK), jnp.int32),
            pltpu.VMEM((NBUF, CHUNK, EMBED_DIM), jnp.float32),
            pltpu.SemaphoreType.DMA((NBUF,)),
            pltpu.SemaphoreType.DMA((NBUF,)),
        ],
        compiler_params=pltpu.CompilerParams(use_tc_tiling_on_sc=False),
    )
    def gather_kernel(idx_hbm, table_hbm, out_hbm, idx_v, rows_v, gsem, ssem):
        wid = lax.axis_index("s") * 2 + lax.axis_index("c")
        base = wid * PER_WORKER

        def start_gather(c):
            b = c % NBUF
            off = base + c * CHUNK
            pltpu.sync_copy(idx_hbm.at[pl.ds(off, CHUNK)], idx_v.at[b])

            def remap_body(i, carry):
                v = idx_v[b, pl.ds(i * 16, 16)]
                idx_v[b, pl.ds(i * 16, 16)] = jnp.where(v == -1, num_item, v)
                return carry

            lax.fori_loop(0, CHUNK // 16, remap_body, 0, unroll=8)
            return pltpu.async_copy(
                table_hbm.at[idx_v.at[b]], rows_v.at[b], gsem.at[b])

        def start_store(c):
            b = c % NBUF
            off = base + c * CHUNK
            return pltpu.async_copy(
                rows_v.at[b], out_hbm.at[pl.ds(off, CHUNK)], ssem.at[b])

        gh = [None] * NUM_CHUNKS
        sh = [None] * NUM_CHUNKS
        for c in range(NUM_CHUNKS):
            if c >= NBUF:
                sh[c - NBUF].wait()
            gh[c] = start_gather(c)
            if c >= 1:
                gh[c - 1].wait()
                sh[c - 1] = start_store(c - 1)
        gh[NUM_CHUNKS - 1].wait()
        sh[NUM_CHUNKS - 1] = start_store(NUM_CHUNKS - 1)
        sh[NUM_CHUNKS - 2].wait()
        sh[NUM_CHUNKS - 1].wait()

    out = gather_kernel(idx_flat, table_lin)
    return out.reshape(BATCH, FIELDS * EMBED_DIM)


# speedup vs baseline: 2.5056x; 1.3121x over previous
import functools

import jax
import jax.numpy as jnp
from jax import lax
from jax.experimental import pallas as pl
from jax.experimental.pallas import tpu as pltpu
from jax.experimental.pallas import tpu_sc as plsc
from jax.experimental.layout import Layout, with_layout_constraint

BATCH = 16384
FIELDS = 26
EMBED_DIM = 32
N = BATCH * FIELDS
NUM_WORKERS = 32
PER_WORKER = N // NUM_WORKERS
CHUNK = 1664
NUM_CHUNKS = PER_WORKER // CHUNK
NBUF = 2

VOCAB1 = 1000001
TBLK = 2048
NBLK = (VOCAB1 + TBLK - 1) // TBLK
VPAD = NBLK * TBLK


def _repack_block(src_ref, dst_ref):
    x = src_ref[...]
    for m in range(TBLK // 512):
        xs = jnp.concatenate(
            [x[:, 512 * m + 128 * a:512 * m + 128 * a + 128] for a in range(4)],
            axis=0)
        dst_ref[128 * m:128 * (m + 1), :] = xs.T


def _repack_table(table):
    tt = table.T
    return pl.pallas_call(
        _repack_block,
        grid=(NBLK,),
        in_specs=[pl.BlockSpec((EMBED_DIM, TBLK), lambda k: (0, k))],
        out_specs=pl.BlockSpec((TBLK // 4, 128), lambda k: (k, 0)),
        out_shape=jax.ShapeDtypeStruct((VPAD // 4, 128), jnp.float32),
    )(tt)


def kernel(indices, table):
    num_item = table.shape[0] - 1
    idx_flat = indices.reshape(-1).astype(jnp.int32)

    t128 = _repack_table(table)
    table_lin = with_layout_constraint(
        t128.reshape(VPAD, EMBED_DIM),
        Layout(major_to_minor=(0, 1), tiling=((8,), (1024,))))

    mesh = plsc.VectorSubcoreMesh(core_axis_name="c", subcore_axis_name="s")

    @functools.partial(
        pl.kernel,
        mesh=mesh,
        out_type=jax.ShapeDtypeStruct((N, EMBED_DIM), jnp.float32),
        scratch_types=[
            pltpu.VMEM((NBUF, CHUNK), jnp.int32),
            pltpu.VMEM((NBUF, CHUNK, EMBED_DIM), jnp.float32),
            pltpu.SemaphoreType.DMA((NBUF,)),
            pltpu.SemaphoreType.DMA((NBUF,)),
        ],
        compiler_params=pltpu.CompilerParams(use_tc_tiling_on_sc=False),
    )
    def gather_kernel(idx_hbm, table_hbm, out_hbm, idx_v, rows_v, gsem, ssem):
        wid = lax.axis_index("s") * 2 + lax.axis_index("c")
        base = wid * PER_WORKER

        def start_gather(c):
            b = c % NBUF
            off = base + c * CHUNK
            pltpu.sync_copy(idx_hbm.at[pl.ds(off, CHUNK)], idx_v.at[b])

            def remap_body(i, carry):
                v = idx_v[b, pl.ds(i * 16, 16)]
                v = jnp.where(v == -1, num_item, v)
                p = ((v & ~511) | ((v & 127) << 2)
                     | ((v >> 7) & 3))
                idx_v[b, pl.ds(i * 16, 16)] = p
                return carry

            lax.fori_loop(0, CHUNK // 16, remap_body, 0, unroll=8)
            return pltpu.async_copy(
                table_hbm.at[idx_v.at[b]], rows_v.at[b], gsem.at[b])

        def start_store(c):
            b = c % NBUF
            off = base + c * CHUNK
            return pltpu.async_copy(
                rows_v.at[b], out_hbm.at[pl.ds(off, CHUNK)], ssem.at[b])

        gh = [None] * NUM_CHUNKS
        sh = [None] * NUM_CHUNKS
        for c in range(NUM_CHUNKS):
            if c >= NBUF:
                sh[c - NBUF].wait()
            gh[c] = start_gather(c)
            if c >= 1:
                gh[c - 1].wait()
                sh[c - 1] = start_store(c - 1)
        gh[NUM_CHUNKS - 1].wait()
        sh[NUM_CHUNKS - 1] = start_store(NUM_CHUNKS - 1)
        sh[NUM_CHUNKS - 2].wait()
        sh[NUM_CHUNKS - 1].wait()

    out = gather_kernel(idx_flat, table_lin)
    return out.reshape(BATCH, FIELDS * EMBED_DIM)


# speedup vs baseline: 4.4837x; 1.7895x over previous
import functools

import jax
import jax.numpy as jnp
from jax import lax
from jax.experimental import pallas as pl
from jax.experimental.pallas import tpu as pltpu
from jax.experimental.pallas import tpu_sc as plsc
from jax.experimental.layout import Layout, with_layout_constraint

BATCH = 16384
FIELDS = 26
EMBED_DIM = 32
N = BATCH * FIELDS
NUM_WORKERS = 32
PER_WORKER = N // NUM_WORKERS
CHUNK = 1664
NUM_CHUNKS = PER_WORKER // CHUNK
NBUF = 2

VOCAB1 = 1000001
TBLK = 16384
NBLK = (VOCAB1 + TBLK - 1) // TBLK
VPAD = NBLK * TBLK


def _repack_block(src_ref, dst_ref):
    x = src_ref[...]
    for m in range(TBLK // 512):
        xs = jnp.concatenate(
            [x[:, 512 * m + 128 * a:512 * m + 128 * a + 128] for a in range(4)],
            axis=0)
        dst_ref[128 * m:128 * (m + 1), :] = xs.T


def _repack_table(table):
    tt = table.T
    return pl.pallas_call(
        _repack_block,
        grid=(NBLK,),
        in_specs=[pl.BlockSpec((EMBED_DIM, TBLK), lambda k: (0, k))],
        out_specs=pl.BlockSpec((TBLK // 4, 128), lambda k: (k, 0)),
        out_shape=jax.ShapeDtypeStruct((VPAD // 4, 128), jnp.float32),
    )(tt)


def kernel(indices, table):
    num_item = table.shape[0] - 1
    idx_flat = indices.reshape(-1).astype(jnp.int32)

    t128 = _repack_table(table)
    table_lin = with_layout_constraint(
        t128.reshape(VPAD, EMBED_DIM),
        Layout(major_to_minor=(0, 1), tiling=((8,), (1024,))))

    mesh = plsc.VectorSubcoreMesh(core_axis_name="c", subcore_axis_name="s")

    @functools.partial(
        pl.kernel,
        mesh=mesh,
        out_type=jax.ShapeDtypeStruct((N, EMBED_DIM), jnp.float32),
        scratch_types=[
            pltpu.VMEM((NBUF, CHUNK), jnp.int32),
            pltpu.VMEM((NBUF, CHUNK, EMBED_DIM), jnp.float32),
            pltpu.SemaphoreType.DMA((NBUF,)),
            pltpu.SemaphoreType.DMA((NBUF,)),
        ],
        compiler_params=pltpu.CompilerParams(use_tc_tiling_on_sc=False),
    )
    def gather_kernel(idx_hbm, table_hbm, out_hbm, idx_v, rows_v, gsem, ssem):
        wid = lax.axis_index("s") * 2 + lax.axis_index("c")
        base = wid * PER_WORKER

        def start_gather(c):
            b = c % NBUF
            off = base + c * CHUNK
            pltpu.sync_copy(idx_hbm.at[pl.ds(off, CHUNK)], idx_v.at[b])

            def remap_body(i, carry):
                v = idx_v[b, pl.ds(i * 16, 16)]
                v = jnp.where(v == -1, num_item, v)
                p = ((v & ~511) | ((v & 127) << 2)
                     | ((v >> 7) & 3))
                idx_v[b, pl.ds(i * 16, 16)] = p
                return carry

            lax.fori_loop(0, CHUNK // 16, remap_body, 0, unroll=8)
            return pltpu.async_copy(
                table_hbm.at[idx_v.at[b]], rows_v.at[b], gsem.at[b])

        def start_store(c):
            b = c % NBUF
            off = base + c * CHUNK
            return pltpu.async_copy(
                rows_v.at[b], out_hbm.at[pl.ds(off, CHUNK)], ssem.at[b])

        gh = [None] * NUM_CHUNKS
        sh = [None] * NUM_CHUNKS
        for c in range(NUM_CHUNKS):
            if c >= NBUF:
                sh[c - NBUF].wait()
            gh[c] = start_gather(c)
            if c >= 1:
                gh[c - 1].wait()
                sh[c - 1] = start_store(c - 1)
        gh[NUM_CHUNKS - 1].wait()
        sh[NUM_CHUNKS - 1] = start_store(NUM_CHUNKS - 1)
        sh[NUM_CHUNKS - 2].wait()
        sh[NUM_CHUNKS - 1].wait()

    out = gather_kernel(idx_flat, table_lin)
    return out.reshape(BATCH, FIELDS * EMBED_DIM)
